# baseline (speedup 1.0000x reference)
import jax
import jax.numpy as jnp
from jax.experimental import pallas as pl
from jax.experimental.pallas import tpu as pltpu

E = 8
NC = 80
REG_MAX = 16
C = 192
B = 16
NO = NC + 4 * REG_MAX
HW0, HW1, HW2 = 4096, 1024, 256
HWT = HW0 + HW1 + HW2


def _moe_kernel(idx_ref, x0_ref, x1_ref, x2_ref, w0_ref, w1_ref, w2_ref,
                c0_ref, c1_ref, c2_ref, out_ref):
    j = pl.program_id(1)

    def dot16(w_ref, x):
        return jnp.dot(w_ref[0].astype(jnp.bfloat16), x.astype(jnp.bfloat16),
                       preferred_element_type=jnp.float32)

    @pl.when(j < 2)
    def _():
        out_ref[0] = dot16(w0_ref, x0_ref[0]) + c0_ref[0]

    @pl.when(j == 2)
    def _():
        out_ref[0, :, 0:HW1] = dot16(w1_ref, x1_ref[0]) + c1_ref[0]
        out_ref[0, :, HW1:HW1 + HW2] = dot16(w2_ref, x2_ref[0]) + c2_ref[0]


def kernel(x0, x1, x2, module_indices, W2_0, b2_0, W3_0, b3_0,
           W2_1, b2_1, W3_1, b3_1, W2_2, b2_2, W3_2, b3_2):
    xs0 = x0.reshape(B, C, HW0)
    xs1 = x1.reshape(B, C, HW1)
    xs2 = x2.reshape(B, C, HW2)
    Ws = [jnp.concatenate([w2, w3], axis=1)
          for w2, w3 in ((W2_0, W3_0), (W2_1, W3_1), (W2_2, W3_2))]
    bs = [jnp.concatenate([bb2, bb3], axis=1)[:, :, None]
          for bb2, bb3 in ((b2_0, b3_0), (b2_1, b3_1), (b2_2, b3_2))]
    idx = module_indices.astype(jnp.int32)

    grid_spec = pltpu.PrefetchScalarGridSpec(
        num_scalar_prefetch=1,
        grid=(B, 3),
        in_specs=[
            pl.BlockSpec((1, C, 2048), lambda b, j, i: (b, 0, jnp.minimum(j, 1))),
            pl.BlockSpec((1, C, HW1), lambda b, j, i: (b, 0, 0)),
            pl.BlockSpec((1, C, HW2), lambda b, j, i: (b, 0, 0)),
            pl.BlockSpec((1, NO, C), lambda b, j, i: (i[b], 0, 0)),
            pl.BlockSpec((1, NO, C), lambda b, j, i: (i[b], 0, 0)),
            pl.BlockSpec((1, NO, C), lambda b, j, i: (i[b], 0, 0)),
            pl.BlockSpec((1, NO, 1), lambda b, j, i: (i[b], 0, 0)),
            pl.BlockSpec((1, NO, 1), lambda b, j, i: (i[b], 0, 0)),
            pl.BlockSpec((1, NO, 1), lambda b, j, i: (i[b], 0, 0)),
        ],
        out_specs=pl.BlockSpec((1, NO, 2048), lambda b, j, i: (b, 0, j)),
    )

    return pl.pallas_call(
        _moe_kernel,
        grid_spec=grid_spec,
        out_shape=jax.ShapeDtypeStruct((B, NO, HWT), jnp.float32),
        compiler_params=pltpu.CompilerParams(
            dimension_semantics=("arbitrary", "arbitrary"),
        ),
    )(idx, xs0, xs1, xs2, Ws[0], Ws[1], Ws[2], bs[0], bs[1], bs[2])


# speedup vs baseline: 1.1472x; 1.1472x over previous
import jax
import jax.numpy as jnp
from jax.experimental import pallas as pl
from jax.experimental.pallas import tpu as pltpu

E = 8
NC = 80
REG_MAX = 16
C = 192
B = 16
NO = NC + 4 * REG_MAX
HW0, HW1, HW2 = 4096, 1024, 256
HWT = HW0 + HW1 + HW2


def _moe_kernel(idx_ref, x0_ref, x1_ref, x2_ref, w0_ref, w1_ref, w2_ref,
                c0_ref, c1_ref, c2_ref, out_ref):
    def dot16(w_ref, x_ref):
        return jnp.dot(w_ref[0].astype(jnp.bfloat16),
                       x_ref[0].astype(jnp.bfloat16),
                       preferred_element_type=jnp.float32)

    out_ref[0, :, 0:HW0] = dot16(w0_ref, x0_ref) + c0_ref[0]
    out_ref[0, :, HW0:HW0 + HW1] = dot16(w1_ref, x1_ref) + c1_ref[0]
    out_ref[0, :, HW0 + HW1:HWT] = dot16(w2_ref, x2_ref) + c2_ref[0]


def kernel(x0, x1, x2, module_indices, W2_0, b2_0, W3_0, b3_0,
           W2_1, b2_1, W3_1, b3_1, W2_2, b2_2, W3_2, b3_2):
    xs0 = x0.reshape(B, C, HW0)
    xs1 = x1.reshape(B, C, HW1)
    xs2 = x2.reshape(B, C, HW2)
    Ws = [jnp.concatenate([w2, w3], axis=1)
          for w2, w3 in ((W2_0, W3_0), (W2_1, W3_1), (W2_2, W3_2))]
    bs = [jnp.concatenate([bb2, bb3], axis=1)[:, :, None]
          for bb2, bb3 in ((b2_0, b3_0), (b2_1, b3_1), (b2_2, b3_2))]
    idx = module_indices.astype(jnp.int32)

    grid_spec = pltpu.PrefetchScalarGridSpec(
        num_scalar_prefetch=1,
        grid=(B,),
        in_specs=[
            pl.BlockSpec((1, C, HW0), lambda b, i: (b, 0, 0)),
            pl.BlockSpec((1, C, HW1), lambda b, i: (b, 0, 0)),
            pl.BlockSpec((1, C, HW2), lambda b, i: (b, 0, 0)),
            pl.BlockSpec((1, NO, C), lambda b, i: (i[b], 0, 0)),
            pl.BlockSpec((1, NO, C), lambda b, i: (i[b], 0, 0)),
            pl.BlockSpec((1, NO, C), lambda b, i: (i[b], 0, 0)),
            pl.BlockSpec((1, NO, 1), lambda b, i: (i[b], 0, 0)),
            pl.BlockSpec((1, NO, 1), lambda b, i: (i[b], 0, 0)),
            pl.BlockSpec((1, NO, 1), lambda b, i: (i[b], 0, 0)),
        ],
        out_specs=pl.BlockSpec((1, NO, HWT), lambda b, i: (b, 0, 0)),
    )

    return pl.pallas_call(
        _moe_kernel,
        grid_spec=grid_spec,
        out_shape=jax.ShapeDtypeStruct((B, NO, HWT), jnp.float32),
        compiler_params=pltpu.CompilerParams(
            dimension_semantics=("parallel",),
        ),
    )(idx, xs0, xs1, xs2, Ws[0], Ws[1], Ws[2], bs[0], bs[1], bs[2])


# speedup vs baseline: 1.1614x; 1.0124x over previous
import jax
import jax.numpy as jnp
from jax.experimental import pallas as pl
from jax.experimental.pallas import tpu as pltpu

E = 8
NC = 80
REG_MAX = 16
C = 192
B = 16
NO = NC + 4 * REG_MAX
HW0, HW1, HW2 = 4096, 1024, 256
HWT = HW0 + HW1 + HW2


def _moe_kernel(idx_ref, x0_ref, x1_ref, x2_ref,
                w0a_ref, w0b_ref, w1a_ref, w1b_ref, w2a_ref, w2b_ref,
                c0a_ref, c0b_ref, c1a_ref, c1b_ref, c2a_ref, c2b_ref,
                out_ref):
    def dot16(w_ref, x):
        return jnp.dot(w_ref[0].astype(jnp.bfloat16), x.astype(jnp.bfloat16),
                       preferred_element_type=jnp.float32)

    for s, (w0, w1, w2, c0, c1, c2) in enumerate((
            (w0a_ref, w1a_ref, w2a_ref, c0a_ref, c1a_ref, c2a_ref),
            (w0b_ref, w1b_ref, w2b_ref, c0b_ref, c1b_ref, c2b_ref))):
        out_ref[s, :, 0:HW0] = dot16(w0, x0_ref[s]) + c0[0]
        out_ref[s, :, HW0:HW0 + HW1] = dot16(w1, x1_ref[s]) + c1[0]
        out_ref[s, :, HW0 + HW1:HWT] = dot16(w2, x2_ref[s]) + c2[0]


def kernel(x0, x1, x2, module_indices, W2_0, b2_0, W3_0, b3_0,
           W2_1, b2_1, W3_1, b3_1, W2_2, b2_2, W3_2, b3_2):
    xs0 = x0.reshape(B, C, HW0)
    xs1 = x1.reshape(B, C, HW1)
    xs2 = x2.reshape(B, C, HW2)
    Ws = [jnp.concatenate([w2, w3], axis=1)
          for w2, w3 in ((W2_0, W3_0), (W2_1, W3_1), (W2_2, W3_2))]
    bs = [jnp.concatenate([bb2, bb3], axis=1)[:, :, None]
          for bb2, bb3 in ((b2_0, b3_0), (b2_1, b3_1), (b2_2, b3_2))]
    idx = module_indices.astype(jnp.int32)

    wspec_a = pl.BlockSpec((1, NO, C), lambda b, i: (i[2 * b], 0, 0))
    wspec_b = pl.BlockSpec((1, NO, C), lambda b, i: (i[2 * b + 1], 0, 0))
    cspec_a = pl.BlockSpec((1, NO, 1), lambda b, i: (i[2 * b], 0, 0))
    cspec_b = pl.BlockSpec((1, NO, 1), lambda b, i: (i[2 * b + 1], 0, 0))

    grid_spec = pltpu.PrefetchScalarGridSpec(
        num_scalar_prefetch=1,
        grid=(B // 2,),
        in_specs=[
            pl.BlockSpec((2, C, HW0), lambda b, i: (b, 0, 0)),
            pl.BlockSpec((2, C, HW1), lambda b, i: (b, 0, 0)),
            pl.BlockSpec((2, C, HW2), lambda b, i: (b, 0, 0)),
            wspec_a, wspec_b, wspec_a, wspec_b, wspec_a, wspec_b,
            cspec_a, cspec_b, cspec_a, cspec_b, cspec_a, cspec_b,
        ],
        out_specs=pl.BlockSpec((2, NO, HWT), lambda b, i: (b, 0, 0)),
    )

    return pl.pallas_call(
        _moe_kernel,
        grid_spec=grid_spec,
        out_shape=jax.ShapeDtypeStruct((B, NO, HWT), jnp.float32),
        compiler_params=pltpu.CompilerParams(
            dimension_semantics=("parallel",),
        ),
    )(idx, xs0, xs1, xs2,
      Ws[0], Ws[0], Ws[1], Ws[1], Ws[2], Ws[2],
      bs[0], bs[0], bs[1], bs[1], bs[2], bs[2])
